# baseline (device time: 91143 ns/iter reference)
import os

import jax
import jax.numpy as jnp
from jax import lax
from jax.experimental import pallas as pl
from jax.experimental.pallas import tpu as pltpu

N_DEV = 4
N_Q = 4
try:
    _MODE = open(os.path.join(os.path.dirname(__file__),
                              "kernel_mode.txt")).read().strip()
except OSError:
    _MODE = "full"


def kernel(x, w_mat, scale_x, scale_w):
    m_global, k_shard = x.shape
    _, n = w_mat.shape
    m_per = m_global // N_DEV
    h = m_per // 2
    qh = h // N_Q

    x = x.astype(jnp.bfloat16)
    w_mat = w_mat.astype(jnp.bfloat16)

    def body(x_ref, w_ref, sx_ref, sw_ref, out_ref,
             send_r, send_l, recv_r, recv_l, send_sems, recv_sems):
        my = lax.axis_index("i")
        left = lax.rem(my + N_DEV - 1, N_DEV)
        right = lax.rem(my + 1, N_DEV)

        barrier_sem = pltpu.get_barrier_semaphore()
        for nbr in (left, right):
            pl.semaphore_signal(
                barrier_sem, inc=1,
                device_id=(nbr,), device_id_type=pl.DeviceIdType.MESH,
            )
        pl.semaphore_wait(barrier_sem, 2)

        def partial(row0, rows, out_dtype):
            xs = x_ref[pl.ds(row0, rows), :]
            acc = jnp.dot(xs, w_ref[...], preferred_element_type=jnp.float32)
            return acc.astype(out_dtype)

        def partial_r(c, out_dtype=jnp.bfloat16):
            return partial(c * m_per, h, out_dtype)

        def partial_l(c, out_dtype=jnp.bfloat16):
            return partial(c * m_per + h, h, out_dtype)

        if _MODE == "nocomm":
            for s in range(N_DEV - 1):
                c_r = lax.rem(my - (s + 1) + N_DEV, N_DEV)
                c_l = lax.rem(my + s + 1, N_DEV)
                for q in range(N_Q):
                    send_r[s % 2, q] = partial_r(c_r)[q * qh:(q + 1) * qh]
                    send_l[s % 2, q] = partial_l(c_l)[q * qh:(q + 1) * qh]
            sc = sx_ref[0] * sw_ref[0]
            p_r = partial_r(my, jnp.float32)
            p_l = partial_l(my, jnp.float32)
            for q in range(N_Q):
                out_ref[pl.ds(q * qh, qh), :] = (
                    p_r[q * qh:(q + 1) * qh]
                    + send_r[0, q].astype(jnp.float32)) * sc
                out_ref[pl.ds(h + q * qh, qh), :] = (
                    p_l[q * qh:(q + 1) * qh]
                    + send_l[0, q].astype(jnp.float32)) * sc
            return

        def start(s, d, q):
            src = send_r if d == 0 else send_l
            dst = recv_r if d == 0 else recv_l
            tgt = right if d == 0 else left
            rdma = pltpu.make_async_remote_copy(
                src_ref=src.at[s % 2, q],
                dst_ref=dst.at[s, q],
                send_sem=send_sems.at[s, d, q],
                recv_sem=recv_sems.at[s, d, q],
                device_id=(tgt,),
                device_id_type=pl.DeviceIdType.MESH,
            )
            rdma.start()
            return rdma

        rdmas = {}
        for s in range(N_DEV - 1):
            c_r = lax.rem(my - (s + 1) + N_DEV, N_DEV)
            c_l = lax.rem(my + s + 1, N_DEV)
            p_r = partial_r(c_r)
            p_l = partial_l(c_l)
            for q in range(N_Q):
                for d, p in ((0, p_r), (1, p_l)):
                    pq = p[q * qh:(q + 1) * qh]
                    if s > 0:
                        rdmas[(s - 1, d, q)].wait_recv()
                        rv = recv_r if d == 0 else recv_l
                        pq = pq + rv[s - 1, q]
                    if s >= 2:
                        rdmas[(s - 2, d, q)].wait_send()
                    sb = send_r if d == 0 else send_l
                    sb[s % 2, q] = pq
                    rdmas[(s, d, q)] = start(s, d, q)

        p_r = partial_r(my, jnp.float32)
        p_l = partial_l(my, jnp.float32)
        sc = sx_ref[0] * sw_ref[0]
        for q in range(N_Q):
            rdmas[(N_DEV - 2, 0, q)].wait_recv()
            out_ref[pl.ds(q * qh, qh), :] = (
                p_r[q * qh:(q + 1) * qh]
                + recv_r[N_DEV - 2, q].astype(jnp.float32)) * sc
            rdmas[(N_DEV - 2, 1, q)].wait_recv()
            out_ref[pl.ds(h + q * qh, qh), :] = (
                p_l[q * qh:(q + 1) * qh]
                + recv_l[N_DEV - 2, q].astype(jnp.float32)) * sc
        for s in range(max(N_DEV - 3, 0), N_DEV - 1):
            for d in range(2):
                for q in range(N_Q):
                    rdmas[(s, d, q)].wait_send()

    return pl.pallas_call(
        body,
        out_shape=jax.ShapeDtypeStruct((m_per, n), jnp.float32),
        in_specs=[
            pl.BlockSpec(memory_space=pltpu.VMEM),
            pl.BlockSpec(memory_space=pltpu.VMEM),
            pl.BlockSpec(memory_space=pltpu.SMEM),
            pl.BlockSpec(memory_space=pltpu.SMEM),
        ],
        out_specs=pl.BlockSpec(memory_space=pltpu.VMEM),
        scratch_shapes=[
            pltpu.VMEM((2, N_Q, qh, n), jnp.bfloat16),
            pltpu.VMEM((2, N_Q, qh, n), jnp.bfloat16),
            pltpu.VMEM((N_DEV - 1, N_Q, qh, n), jnp.bfloat16),
            pltpu.VMEM((N_DEV - 1, N_Q, qh, n), jnp.bfloat16),
            pltpu.SemaphoreType.DMA((N_DEV - 1, 2, N_Q)),
            pltpu.SemaphoreType.DMA((N_DEV - 1, 2, N_Q)),
        ],
        compiler_params=pltpu.CompilerParams(collective_id=0),
    )(x, w_mat, scale_x, scale_w)


# device time: 83504 ns/iter; 1.0915x vs baseline; 1.0915x over previous
import jax
import jax.numpy as jnp
from jax import lax
from jax.experimental import pallas as pl
from jax.experimental.pallas import tpu as pltpu

N_DEV = 4
N_Q = 4
N_ST = 3


def kernel(x, w_mat, scale_x, scale_w):
    m_global, k_shard = x.shape
    _, n = w_mat.shape
    m_per = m_global // N_DEV
    h = m_per // 2
    qh = h // N_Q

    w_mat = w_mat.astype(jnp.bfloat16)

    def body(x_ref, w_ref, sx_ref, sw_ref, out_ref,
             xst, send_r, send_l, recv_r, recv_l,
             copy_sems, send_sems, recv_sems):
        my = lax.axis_index("i")
        left = lax.rem(my + N_DEV - 1, N_DEV)
        right = lax.rem(my + 1, N_DEV)

        barrier_sem = pltpu.get_barrier_semaphore()
        for nbr in (left, right):
            pl.semaphore_signal(
                barrier_sem, inc=1,
                device_id=(nbr,), device_id_type=pl.DeviceIdType.MESH,
            )
        pl.semaphore_wait(barrier_sem, 2)

        rows = []
        for s in range(N_DEV - 1):
            c_r = lax.rem(my - (s + 1) + N_DEV, N_DEV)
            c_l = lax.rem(my + s + 1, N_DEV)
            for q in range(N_Q):
                rows.append(c_r * m_per + q * qh)
                rows.append(c_l * m_per + h + q * qh)
        for q in range(N_Q):
            rows.append(my * m_per + q * qh)
            rows.append(my * m_per + h + q * qh)

        copies = {}

        def issue_copy(i):
            if i >= len(rows):
                return
            cp = pltpu.make_async_copy(
                x_ref.at[pl.ds(rows[i], qh), :],
                xst.at[i % N_ST],
                copy_sems.at[i % N_ST],
            )
            cp.start()
            copies[i] = cp

        def staged_gemm(i):
            copies[i].wait()
            xs = xst[i % N_ST].astype(jnp.bfloat16)
            acc = jnp.dot(xs, w_ref[...], preferred_element_type=jnp.float32)
            issue_copy(i + 2)
            return acc

        issue_copy(0)
        issue_copy(1)

        def start(s, d, q):
            src = send_r if d == 0 else send_l
            dst = recv_r if d == 0 else recv_l
            tgt = right if d == 0 else left
            rdma = pltpu.make_async_remote_copy(
                src_ref=src.at[s % 2, q],
                dst_ref=dst.at[s, q],
                send_sem=send_sems.at[s, d, q],
                recv_sem=recv_sems.at[s, d, q],
                device_id=(tgt,),
                device_id_type=pl.DeviceIdType.MESH,
            )
            rdma.start()
            return rdma

        site = 0
        rdmas = {}
        for s in range(N_DEV - 1):
            for q in range(N_Q):
                for d in range(2):
                    pq = staged_gemm(site).astype(jnp.bfloat16)
                    site += 1
                    if s > 0:
                        rdmas[(s - 1, d, q)].wait_recv()
                        rv = recv_r if d == 0 else recv_l
                        pq = pq + rv[s - 1, q]
                    if s >= 2:
                        rdmas[(s - 2, d, q)].wait_send()
                    sb = send_r if d == 0 else send_l
                    sb[s % 2, q] = pq
                    rdmas[(s, d, q)] = start(s, d, q)

        sc = sx_ref[0] * sw_ref[0]
        for q in range(N_Q):
            for d in range(2):
                acc = staged_gemm(site)
                site += 1
                rdmas[(N_DEV - 2, d, q)].wait_recv()
                rv = recv_r if d == 0 else recv_l
                out_ref[pl.ds(d * h + q * qh, qh), :] = (
                    acc + rv[N_DEV - 2, q].astype(jnp.float32)) * sc
        for s in range(max(N_DEV - 3, 0), N_DEV - 1):
            for d in range(2):
                for q in range(N_Q):
                    rdmas[(s, d, q)].wait_send()

    return pl.pallas_call(
        body,
        out_shape=jax.ShapeDtypeStruct((m_per, n), jnp.float32),
        in_specs=[
            pl.BlockSpec(memory_space=pltpu.MemorySpace.HBM),
            pl.BlockSpec(memory_space=pltpu.VMEM),
            pl.BlockSpec(memory_space=pltpu.SMEM),
            pl.BlockSpec(memory_space=pltpu.SMEM),
        ],
        out_specs=pl.BlockSpec(memory_space=pltpu.VMEM),
        scratch_shapes=[
            pltpu.VMEM((N_ST, qh, k_shard), jnp.float32),
            pltpu.VMEM((2, N_Q, qh, n), jnp.bfloat16),
            pltpu.VMEM((2, N_Q, qh, n), jnp.bfloat16),
            pltpu.VMEM((N_DEV - 1, N_Q, qh, n), jnp.bfloat16),
            pltpu.VMEM((N_DEV - 1, N_Q, qh, n), jnp.bfloat16),
            pltpu.SemaphoreType.DMA((N_ST,)),
            pltpu.SemaphoreType.DMA((N_DEV - 1, 2, N_Q)),
            pltpu.SemaphoreType.DMA((N_DEV - 1, 2, N_Q)),
        ],
        compiler_params=pltpu.CompilerParams(collective_id=0),
    )(x, w_mat, scale_x, scale_w)
